# baseline (speedup 1.0000x reference)
import functools

import jax
import jax.numpy as jnp
from jax import lax
from jax.experimental import pallas as pl
from jax.experimental.pallas import tpu as pltpu
from jax.experimental.pallas import tpu_sc as plsc

QUES = 2048
SEQH = 2048
T = 8192
K = 32

G1 = 32
QROWS = T // G1
WROWS = (3 * SEQH) // G1

NW = 16
CH = T // NW
NV = CH // 16
NEG = -3.0e38


def _matvec_body(sel_ref, q_ref, h_ref, ques_ref, wih_ref, whh_ref,
                 alpha_ref, gi_ref, gh_ref):
    q = q_ref[...]
    h = h_ref[...]
    alpha_ref[...] = jnp.dot(ques_ref[...], q,
                             preferred_element_type=jnp.float32)
    gi_ref[...] = jnp.dot(wih_ref[...], q,
                          preferred_element_type=jnp.float32)
    gh_ref[...] = jnp.dot(whh_ref[...], h,
                          preferred_element_type=jnp.float32)


def _sc_attn_body(alpha, hs2, ws2, q, bs, pred_out,
                  avv, lvv, liv, cvv, civ, gvv, giv,
                  idx1, zidx, row_v, acc, bufa, bufb, qdb, bs_v, predb,
                  sv_sh, si_sh, w_sh, i_sh, qd_sh, attn_sh, sem):
    cid = lax.axis_index("c")
    sid = lax.axis_index("s")
    lane = lax.broadcasted_iota(jnp.int32, (16,), 0)
    m0 = lane == 0
    zf = jnp.zeros((16,), jnp.float32)
    zi = jnp.zeros((16,), jnp.int32)

    @pl.when(cid == 0)
    def _():
        base = pl.multiple_of(sid * CH, CH)
        pltpu.sync_copy(alpha.at[pl.ds(base, CH)], avv)

        def fold(t, carry):
            vb, ib = carry
            v = avv[pl.ds(t * 16, 16)]
            gidx = base + t * 16 + lane
            better = v > vb
            return jnp.where(better, v, vb), jnp.where(better, gidx, ib)

        def ex(p, _):
            vb, ib = lax.fori_loop(1, NV, fold, (avv[pl.ds(0, 16)],
                                                 base + lane))
            mval = jnp.max(vb)
            midx = jnp.max(jnp.where(vb == mval, ib, jnp.int32(-1)))
            pv = zi + p
            plsc.store_scatter(lvv, [pv], zf + mval, mask=m0)
            plsc.store_scatter(liv, [pv], zi + midx, mask=m0)
            plsc.store_scatter(avv, [zi + (midx - base)], zf + NEG, mask=m0)
            return 0

        lax.fori_loop(0, K, ex, 0)
        pltpu.sync_copy(lvv, sv_sh.at[pl.ds(sid * K, K)])
        pltpu.sync_copy(liv, si_sh.at[pl.ds(sid * K, K)])

        @pl.when(sid == 1)
        def _():
            pltpu.sync_copy(ws2.at[0], bufa)
            pltpu.sync_copy(q, bufb)

            def dstep(t, c):
                return c + bufa[pl.ds(t * 16, 16)] * bufb[pl.ds(t * 16, 16)]

            qdb[...] = lax.fori_loop(0, QUES // 16, dstep, zf)
            pltpu.sync_copy(qdb, qd_sh)

        plsc.subcore_barrier()

        @pl.when(sid == 0)
        def _():
            pltpu.sync_copy(sv_sh, cvv)
            pltpu.sync_copy(si_sh, civ)

            def gfold(t, carry):
                vb, ib = carry
                v = cvv[pl.ds(t * 16, 16)]
                slot = t * 16 + lane
                better = v > vb
                return jnp.where(better, v, vb), jnp.where(better, slot, ib)

            def gex(p, _):
                vb, ib = lax.fori_loop(1, NW * K // 16, gfold,
                                       (cvv[pl.ds(0, 16)], lane))
                mval = jnp.max(vb)
                mslot = jnp.max(jnp.where(vb == mval, ib, jnp.int32(-1)))
                slotv = zi + mslot
                orig = plsc.load_gather(civ, [slotv])
                pv = zi + p
                plsc.store_scatter(gvv, [pv], zf + mval, mask=m0)
                plsc.store_scatter(giv, [pv], orig, mask=m0)
                plsc.store_scatter(cvv, [slotv], zf + NEG, mask=m0)
                return 0

            lax.fori_loop(0, K, gex, 0)
            v0 = gvv[pl.ds(0, 16)]
            v1 = gvv[pl.ds(16, 16)]
            mx = jnp.maximum(jnp.max(v0), jnp.max(v1))
            e0 = jnp.exp(v0 - mx)
            e1 = jnp.exp(v1 - mx)
            inv = (zf + 1.0) / (zf + (jnp.sum(e0) + jnp.sum(e1)))
            gvv[pl.ds(0, 16)] = e0 * inv
            gvv[pl.ds(16, 16)] = e1 * inv
            pltpu.sync_copy(gvv, w_sh)
            pltpu.sync_copy(giv, i_sh)

            def zstep(t, _):
                acc.at[0][pl.ds(t * 16, 16)] = zf
                return 0

            lax.fori_loop(0, SEQH // 16, zstep, 0)
            pltpu.sync_copy(acc.at[0], attn_sh.at[0])

        plsc.subcore_barrier()

        pltpu.sync_copy(w_sh, lvv)
        pltpu.sync_copy(i_sh, liv)
        plsc.store_scatter(zidx, [zi], zi, mask=m0)

        def zstep2(t, _):
            acc.at[0][pl.ds(t * 16, 16)] = zf
            return 0

        lax.fori_loop(0, SEQH // 16, zstep2, 0)
        for r in range(2):
            j = sid * 2 + r
            jv = zi + j
            myi = plsc.load_gather(liv, [jv])
            myw = plsc.load_gather(lvv, [jv])
            plsc.store_scatter(idx1, [zi], myi, mask=m0)
            pltpu.async_copy(hs2.at[idx1], row_v, sem).wait()

            def wstep(t, _, w=myw):
                s = pl.ds(t * 16, 16)
                acc.at[0][s] = acc.at[0][s] + row_v.at[0][s] * w
                return 0

            lax.fori_loop(0, SEQH // 16, wstep, 0)
        pltpu.sync_copy(acc, attn_sh.at[zidx], add=True)
        plsc.subcore_barrier()

        @pl.when(sid == 0)
        def _():
            pltpu.sync_copy(attn_sh.at[0], bufa)
            pltpu.sync_copy(ws2.at[1], bufb)

            def dstep2(t, c):
                return c + bufa[pl.ds(t * 16, 16)] * bufb[pl.ds(t * 16, 16)]

            hv = lax.fori_loop(0, SEQH // 16, dstep2, zf)
            pltpu.sync_copy(qd_sh, qdb)
            pltpu.sync_copy(bs, bs_v)
            bsv = plsc.load_gather(bs_v, [zi])
            total = zf + (jnp.sum(hv) + jnp.sum(qdb[...])) + bsv
            plsc.store_scatter(predb, [zi], total, mask=m0)
            pltpu.sync_copy(predb, pred_out)


def _sc_attn(alpha, hs2, ws2, q, bs):
    f32 = jnp.float32
    i32 = jnp.int32
    mesh = plsc.VectorSubcoreMesh(core_axis_name="c", subcore_axis_name="s")
    return pl.kernel(
        _sc_attn_body,
        mesh=mesh,
        compiler_params=pltpu.CompilerParams(needs_layout_passes=False),
        out_type=jax.ShapeDtypeStruct((1,), f32),
        scratch_types=[
            pltpu.VMEM((CH,), f32),
            pltpu.VMEM((K,), f32),
            pltpu.VMEM((K,), i32),
            pltpu.VMEM((NW * K,), f32),
            pltpu.VMEM((NW * K,), i32),
            pltpu.VMEM((K,), f32),
            pltpu.VMEM((K,), i32),
            pltpu.VMEM((1,), i32),
            pltpu.VMEM((1,), i32),
            pltpu.VMEM((1, SEQH), f32),
            pltpu.VMEM((1, SEQH), f32),
            pltpu.VMEM((QUES,), f32),
            pltpu.VMEM((QUES,), f32),
            pltpu.VMEM((16,), f32),
            pltpu.VMEM((1,), f32),
            pltpu.VMEM((1,), f32),
            pltpu.VMEM_SHARED((NW * K,), f32),
            pltpu.VMEM_SHARED((NW * K,), i32),
            pltpu.VMEM_SHARED((K,), f32),
            pltpu.VMEM_SHARED((K,), i32),
            pltpu.VMEM_SHARED((16,), f32),
            pltpu.VMEM_SHARED((1, SEQH), f32),
            pltpu.SemaphoreType.DMA,
        ],
    )(alpha, hs2, ws2, q, bs)


def _combine_body(gi_ref, gh_ref, h_ref, bih_ref, bhh_ref, hnew_ref):
    gi = gi_ref[...] + bih_ref[...]
    gh = gh_ref[...] + bhh_ref[...]
    h = h_ref[...]
    i_r, i_z, i_n = gi[0:16], gi[16:32], gi[32:48]
    h_r, h_z, h_n = gh[0:16], gh[16:32], gh[32:48]
    r = jax.nn.sigmoid(i_r + h_r)
    z = jax.nn.sigmoid(i_z + h_z)
    n = jnp.tanh(i_n + r * h_n)
    hnew_ref[...] = (1.0 - z) * n + z * h


def kernel(question, score, questions, hs, Ws, bs, W_ih, W_hh, b_ih, b_hh):
    f32 = jnp.float32
    q2 = question.reshape(QUES, 1)
    h_prev = hs[T - 1, 0]
    h2 = h_prev.reshape(SEQH, 1)
    sel = (score[0] < 0.5).astype(jnp.int32).reshape(1)

    grid_spec = pltpu.PrefetchScalarGridSpec(
        num_scalar_prefetch=1,
        grid=(G1,),
        in_specs=[
            pl.BlockSpec((QUES, 1), lambda i, s: (0, 0)),
            pl.BlockSpec((SEQH, 1), lambda i, s: (0, 0)),
            pl.BlockSpec((QROWS, QUES), lambda i, s: (i, 0)),
            pl.BlockSpec((WROWS, QUES), lambda i, s: (i, s[0])),
            pl.BlockSpec((WROWS, SEQH), lambda i, s: (i, 0)),
        ],
        out_specs=[
            pl.BlockSpec((QROWS, 1), lambda i, s: (i, 0)),
            pl.BlockSpec((WROWS, 1), lambda i, s: (i, 0)),
            pl.BlockSpec((WROWS, 1), lambda i, s: (i, 0)),
        ],
    )
    alpha, gi, gh = pl.pallas_call(
        _matvec_body,
        grid_spec=grid_spec,
        out_shape=[
            jax.ShapeDtypeStruct((T, 1), f32),
            jax.ShapeDtypeStruct((3 * SEQH, 1), f32),
            jax.ShapeDtypeStruct((3 * SEQH, 1), f32),
        ],
    )(sel, q2, h2, questions, W_ih, W_hh)

    pred = _sc_attn(alpha.reshape(T), hs.reshape(T, SEQH),
                    Ws.reshape(2, QUES), question, bs)

    h_new = pl.pallas_call(
        _combine_body,
        out_shape=jax.ShapeDtypeStruct((16, 128), f32),
    )(
        gi.reshape(48, 128), gh.reshape(48, 128), h_prev.reshape(16, 128),
        b_ih.reshape(48, 128), b_hh.reshape(48, 128),
    )
    return (pred, h_new.reshape(1, 1, SEQH))
